# baseline (speedup 1.0000x reference)
import functools

import jax
import jax.numpy as jnp
from jax import lax
from jax.experimental import pallas as pl
from jax.experimental.pallas import tpu as pltpu
from jax.experimental.pallas import tpu_sc as plsc

_NUM_LEVELS = 8
_VOCAB = 1024
_EMBED_DIM = 64

_C = 128
_G = 2
_M = _C * _G
_NBUF = 4


@functools.lru_cache(maxsize=None)
def _build(num_rows):
    info = plsc.get_sparse_core_info()
    nc, ns = info.num_cores, info.num_subcores
    nw = nc * ns
    rows_per_w = num_rows // nw
    idx_rows_per_w = rows_per_w // _C
    nm = rows_per_w // _M
    assert nm >= 4 and (nm - 4) % _NBUF == 0

    mesh = plsc.VectorSubcoreMesh(core_axis_name="c", subcore_axis_name="s")

    @functools.partial(
        pl.kernel,
        mesh=mesh,
        out_type=jax.ShapeDtypeStruct((num_rows, _EMBED_DIM), jnp.float32),
        compiler_params=pltpu.CompilerParams(use_tc_tiling_on_sc=False),
        scratch_types=[
            pltpu.VMEM((idx_rows_per_w, _C), jnp.int32),
            pltpu.VMEM((_NBUF, _M, _EMBED_DIM), jnp.float32),
        ]
        + [pltpu.SemaphoreType.DMA] * (2 * _NBUF),
    )
    def k(idx_hbm, table_hbm, out_hbm, idx_v, rows_v, *sems):
        gsem = sems[:_NBUF]
        wsem = sems[_NBUF:]
        wid = lax.axis_index("s") * nc + lax.axis_index("c")
        row_base = wid * rows_per_w

        pltpu.sync_copy(
            idx_hbm.at[pl.ds(wid * idx_rows_per_w, idx_rows_per_w)], idx_v
        )

        def g_descs(mc, buf):
            return [
                pltpu.make_async_copy(
                    table_hbm.at[idx_v.at[mc * _G + g]],
                    rows_v.at[buf, pl.ds(g * _C, _C)],
                    gsem[buf],
                )
                for g in range(_G)
            ]

        def w_desc(mc, buf):
            return pltpu.make_async_copy(
                rows_v.at[buf],
                out_hbm.at[pl.ds(row_base + mc * _M, _M)],
                wsem[buf],
            )

        def start_g(mc, buf):
            for d in g_descs(mc, buf):
                d.start()

        def wait_g(mc, buf):
            for d in g_descs(mc, buf):
                d.wait()

        for mc in range(_NBUF):
            start_g(mc, mc)
        wait_g(0, 0)
        w_desc(0, 0).start()
        wait_g(1, 1)
        w_desc(1, 1).start()

        def body(j, carry):
            for b4 in range(_NBUF):
                mc = 2 + j * _NBUF + b4
                buf = (2 + b4) % _NBUF
                nbuf = b4 % _NBUF
                wait_g(mc, buf)
                w_desc(mc, buf).start()
                w_desc(mc - 2, nbuf).wait()
                start_g(mc + 2, nbuf)
            return carry

        lax.fori_loop(0, (nm - 4) // _NBUF, body, 0)

        for mc in (nm - 2, nm - 1):
            buf = mc % _NBUF
            wait_g(mc, buf)
            w_desc(mc, buf).start()
            w_desc(mc - 2, (mc - 2) % _NBUF).wait()
        w_desc(nm - 2, (nm - 2) % _NBUF).wait()
        w_desc(nm - 1, (nm - 1) % _NBUF).wait()

    return k


def kernel(codes, tables):
    b, l, q = codes.shape
    _, v, d = tables.shape
    n = b * l * q
    adj = codes + jnp.arange(q, dtype=codes.dtype) * v
    adj = adj.reshape(b, l // 8, 8, q // 2, 2)
    adj = adj.transpose(0, 1, 3, 2, 4)
    idx = adj.reshape(n // _C, _C)
    out = _build(n)(idx, tables.reshape(q * v, d))
    out = out.reshape(b, l // 8, 4, 8, 2, d)
    out = out.transpose(0, 1, 3, 2, 4, 5)
    return out.reshape(b, l, q * d)


# speedup vs baseline: 5.5679x; 5.5679x over previous
import functools

import jax
import jax.numpy as jnp
from jax import lax
from jax.experimental import pallas as pl
from jax.experimental.pallas import tpu as pltpu
from jax.experimental.pallas import tpu_sc as plsc

_NUM_LEVELS = 8
_VOCAB = 1024
_EMBED_DIM = 64

_C = 128
_G = 2
_M = _C * _G
_NBUF = 4


@functools.lru_cache(maxsize=None)
def _build(num_rows):
    info = plsc.get_sparse_core_info()
    nc, ns = info.num_cores, info.num_subcores
    nw = nc * ns
    rows_per_w = num_rows // nw
    idx_rows_per_w = rows_per_w // _C
    nm = rows_per_w // _M
    assert nm >= 4 and (nm - 4) % _NBUF == 0

    mesh = plsc.VectorSubcoreMesh(core_axis_name="c", subcore_axis_name="s")

    @functools.partial(
        pl.kernel,
        mesh=mesh,
        out_type=jax.ShapeDtypeStruct((num_rows, _EMBED_DIM), jnp.float32),
        compiler_params=pltpu.CompilerParams(use_tc_tiling_on_sc=False),
        scratch_types=[
            pltpu.VMEM((idx_rows_per_w, _C), jnp.int32),
            pltpu.VMEM((_NBUF, _M, _EMBED_DIM), jnp.float32),
        ]
        + [pltpu.SemaphoreType.DMA] * (2 * _NBUF),
    )
    def k(idx_hbm, table_hbm, out_hbm, idx_v, rows_v, *sems):
        gsem = sems[:_NBUF]
        wsem = sems[_NBUF:]
        wid = lax.axis_index("s") * nc + lax.axis_index("c")
        row_base = wid * rows_per_w

        pltpu.sync_copy(
            idx_hbm.at[pl.ds(wid * idx_rows_per_w, idx_rows_per_w)], idx_v
        )

        def g_descs(mc, buf):
            return [
                pltpu.make_async_copy(
                    table_hbm.at[idx_v.at[mc * _G + g]],
                    rows_v.at[buf, pl.ds(g * _C, _C)],
                    gsem[buf],
                )
                for g in range(_G)
            ]

        def w_desc(mc, buf):
            return pltpu.make_async_copy(
                rows_v.at[buf],
                out_hbm.at[pl.ds(row_base + mc * _M, _M)],
                wsem[buf],
            )

        def start_g(mc, buf):
            for d in g_descs(mc, buf):
                d.start()

        def wait_g(mc, buf):
            for d in g_descs(mc, buf):
                d.wait()

        for mc in range(_NBUF):
            start_g(mc, mc)
        wait_g(0, 0)
        w_desc(0, 0).start()
        wait_g(1, 1)
        w_desc(1, 1).start()

        def body(j, carry):
            for b4 in range(_NBUF):
                mc = 2 + j * _NBUF + b4
                buf = (2 + b4) % _NBUF
                nbuf = b4 % _NBUF
                wait_g(mc, buf)
                w_desc(mc, buf).start()
                w_desc(mc - 2, nbuf).wait()
                start_g(mc + 2, nbuf)
            return carry

        lax.fori_loop(0, (nm - 4) // _NBUF, body, 0)

        for mc in (nm - 2, nm - 1):
            buf = mc % _NBUF
            wait_g(mc, buf)
            w_desc(mc, buf).start()
            w_desc(mc - 2, (mc - 2) % _NBUF).wait()
        w_desc(nm - 2, (nm - 2) % _NBUF).wait()
        w_desc(nm - 1, (nm - 1) % _NBUF).wait()

    return k


def kernel(codes, tables):
    b, l, q = codes.shape
    _, v, d = tables.shape
    n = b * l * q
    adj = codes + jnp.arange(q, dtype=codes.dtype) * v
    idx = adj.reshape(n // _C, _C)
    out = _build(n)(idx, tables.reshape(q * v, d))
    return out.reshape(b, l, q * d)
